# baseline (speedup 1.0000x reference)
import functools

import jax
import jax.numpy as jnp
from jax import lax
from jax.experimental import pallas as pl
from jax.experimental.pallas import tpu as pltpu
from jax.experimental.pallas import tpu_sc as plsc

D = 128
LANES = 16
NUM_CORES = 2
NUM_SUBCORES = 16
NW = NUM_CORES * NUM_SUBCORES
CHUNK = 64
GRP = 16


def _dotT(x, w):
    return lax.dot_general(x, w, (((1,), (1,)), ((), ())),
                           preferred_element_type=jnp.float32)


def _tc_project2(x, wa, wb, bias_b, blk, n):
    grid = n // blk

    def body(x_ref, wa_ref, wb_ref, b_ref, oa_ref, ob_ref):
        x_ = x_ref[...]
        oa_ref[...] = _dotT(x_, wa_ref[...])
        ob_ref[...] = _dotT(x_, wb_ref[...]) + b_ref[...]

    return pl.pallas_call(
        body,
        grid=(grid,),
        in_specs=[
            pl.BlockSpec((blk, D), lambda i: (i, 0)),
            pl.BlockSpec((D, D), lambda i: (0, 0)),
            pl.BlockSpec((D, D), lambda i: (0, 0)),
            pl.BlockSpec((1, D), lambda i: (0, 0)),
        ],
        out_specs=[
            pl.BlockSpec((blk, D), lambda i: (i, 0)),
            pl.BlockSpec((blk, D), lambda i: (i, 0)),
        ],
        out_shape=[
            jax.ShapeDtypeStruct((n, D), jnp.float32),
            jax.ShapeDtypeStruct((n, D), jnp.float32),
        ],
    )(x, wa, wb, bias_b)


def _tc_sig_project2(parts, xr, wa, wb, bias_b, blk, n):
    grid = n // blk

    def body(p_ref, xr_ref, wa_ref, wb_ref, b_ref, oa_ref, ob_ref):
        h = jax.nn.sigmoid(p_ref[0] + p_ref[1] + xr_ref[...])
        oa_ref[...] = _dotT(h, wa_ref[...])
        ob_ref[...] = _dotT(h, wb_ref[...]) + b_ref[...]

    return pl.pallas_call(
        body,
        grid=(grid,),
        in_specs=[
            pl.BlockSpec((NUM_CORES, blk, D), lambda i: (0, i, 0)),
            pl.BlockSpec((blk, D), lambda i: (i, 0)),
            pl.BlockSpec((D, D), lambda i: (0, 0)),
            pl.BlockSpec((D, D), lambda i: (0, 0)),
            pl.BlockSpec((1, D), lambda i: (0, 0)),
        ],
        out_specs=[
            pl.BlockSpec((blk, D), lambda i: (i, 0)),
            pl.BlockSpec((blk, D), lambda i: (i, 0)),
        ],
        out_shape=[
            jax.ShapeDtypeStruct((n, D), jnp.float32),
            jax.ShapeDtypeStruct((n, D), jnp.float32),
        ],
    )(parts, xr, wa, wb, bias_b)


def _tc_sig_sum(parts, hr, blk, n):
    grid = n // blk

    def body(p_ref, hr_ref, o_ref):
        o_ref[...] = jax.nn.sigmoid(p_ref[0] + p_ref[1] + hr_ref[...])

    return pl.pallas_call(
        body,
        grid=(grid,),
        in_specs=[
            pl.BlockSpec((NUM_CORES, blk, D), lambda i: (0, i, 0)),
            pl.BlockSpec((blk, D), lambda i: (i, 0)),
        ],
        out_specs=pl.BlockSpec((blk, D), lambda i: (i, 0)),
        out_shape=jax.ShapeDtypeStruct((n, D), jnp.float32),
    )(parts, hr)


def _sc_segment_sum(g, src_r, dst_r, w_r, n, ngrp):
    rows_per_sub = (n // NUM_SUBCORES) // 8 * 8
    rem_rows = n - NUM_SUBCORES * rows_per_sub
    mesh = plsc.VectorSubcoreMesh(core_axis_name="c", subcore_axis_name="s")

    @functools.partial(
        pl.kernel,
        mesh=mesh,
        out_type=jax.ShapeDtypeStruct((NUM_CORES, n, D), jnp.float32),
        scratch_types=[
            pltpu.VMEM((GRP, CHUNK), jnp.int32),
            pltpu.VMEM((GRP, CHUNK), jnp.int32),
            pltpu.VMEM((GRP * CHUNK,), jnp.float32),
            pltpu.VMEM((CHUNK, D), jnp.float32),
            pltpu.VMEM_SHARED((n, D), jnp.float32),
            pltpu.SemaphoreType.DMA,
        ],
    )
    def k(g_hbm, src_hbm, dst_hbm, w_hbm, out_hbm,
          src_v, dst_v, w_v, rows_v, agg_s, sem):
        cid = lax.axis_index("c")
        sid = lax.axis_index("s")
        wid = sid * NUM_CORES + cid

        def zbody(i, carry):
            for j in range(D // LANES):
                rows_v[i, pl.ds(j * LANES, LANES)] = jnp.zeros((LANES,), jnp.float32)
            return carry
        lax.fori_loop(0, CHUNK, zbody, 0)
        zbase = sid * rows_per_sub
        done = 0
        while done < rows_per_sub:
            step = min(CHUNK, rows_per_sub - done)
            pltpu.sync_copy(rows_v.at[pl.ds(0, step)],
                            agg_s.at[pl.ds(zbase + done, step)])
            done += step
        if rem_rows:
            @pl.when(sid == NUM_SUBCORES - 1)
            def _zero_tail():
                pltpu.sync_copy(
                    rows_v.at[pl.ds(0, rem_rows)],
                    agg_s.at[pl.ds(NUM_SUBCORES * rows_per_sub, rem_rows)])
        plsc.subcore_barrier()

        def grp_body(gg, carry):
            pltpu.sync_copy(src_hbm.at[wid, gg], src_v)
            pltpu.sync_copy(dst_hbm.at[wid, gg], dst_v)
            pltpu.sync_copy(w_hbm.at[wid, gg], w_v)

            def chunk_body(kk, c1):
                pltpu.async_copy(g_hbm.at[src_v.at[kk]], rows_v, sem).wait()

                def gbody(gi, c2):
                    w16 = w_v[pl.ds(kk * CHUNK + gi * LANES, LANES)]
                    for i in range(LANES):
                        w_splat = jnp.full((LANES,), w16[i], jnp.float32)
                        ei = gi * LANES + i
                        for j in range(D // LANES):
                            sl = pl.ds(j * LANES, LANES)
                            rows_v[ei, sl] = rows_v[ei, sl] * w_splat
                    return c2
                lax.fori_loop(0, CHUNK // LANES, gbody, 0)

                pltpu.sync_copy(rows_v, agg_s.at[dst_v.at[kk]], add=True)
                return c1
            lax.fori_loop(0, GRP, chunk_body, 0)
            return carry
        lax.fori_loop(0, ngrp, grp_body, 0)
        plsc.subcore_barrier()

        pltpu.sync_copy(agg_s.at[pl.ds(zbase, rows_per_sub)],
                        out_hbm.at[cid, pl.ds(zbase, rows_per_sub)])
        if rem_rows:
            @pl.when(sid == NUM_SUBCORES - 1)
            def _pub_tail():
                tb = NUM_SUBCORES * rows_per_sub
                pltpu.sync_copy(agg_s.at[pl.ds(tb, rem_rows)],
                                out_hbm.at[cid, pl.ds(tb, rem_rows)])

    return k(g, src_r, dst_r, w_r)


def kernel(X, edge_index, edge_weight,
           W_rel1, b_rel1, W_root1, b_root1,
           W_rel2, b_rel2, W_root2, b_root2):
    n = X.shape[0]
    e = edge_weight.shape[0]
    chunks_pt = -(-(-(-e // NW)) // CHUNK)
    chunks_pt = -(-chunks_pt // GRP) * GRP
    ngrp = chunks_pt // GRP
    e_pad = NW * chunks_pt * CHUNK
    pad = e_pad - e
    src = jnp.concatenate([edge_index[0], jnp.zeros((pad,), edge_index.dtype)])
    dst = jnp.concatenate([edge_index[1], jnp.zeros((pad,), edge_index.dtype)])
    wf = jnp.concatenate([edge_weight, jnp.zeros((pad,), edge_weight.dtype)])

    blk = n // 10

    src_r = src.reshape(NW, ngrp, GRP, CHUNK)
    dst_r = dst.reshape(NW, ngrp, GRP, CHUNK)
    w_r = wf.reshape(NW, ngrp, GRP * CHUNK)
    b1 = (b_rel1 + b_root1).reshape(1, D)
    b2 = (b_rel2 + b_root2).reshape(1, D)

    xw1, xr1 = _tc_project2(X, W_rel1, W_root1, b1, blk, n)
    parts1 = _sc_segment_sum(xw1, src_r, dst_r, w_r, n, ngrp)
    hw2, hr2 = _tc_sig_project2(parts1, xr1, W_rel2, W_root2, b2, blk, n)
    parts2 = _sc_segment_sum(hw2, src_r, dst_r, w_r, n, ngrp)
    return _tc_sig_sum(parts2, hr2, blk, n)


# speedup vs baseline: 2.0903x; 2.0903x over previous
import functools

import jax
import jax.numpy as jnp
from jax import lax
from jax.experimental import pallas as pl
from jax.experimental.pallas import tpu as pltpu
from jax.experimental.pallas import tpu_sc as plsc

D = 128
LANES = 16
NUM_CORES = 2
NUM_SUBCORES = 16
NW = NUM_CORES * NUM_SUBCORES
CHUNK = 64
GRP = 16


def _dotT(x, w):
    return lax.dot_general(x, w, (((1,), (1,)), ((), ())),
                           preferred_element_type=jnp.float32)


def _tc_project2(x, wa, wb, bias_b, blk, n):
    grid = n // blk

    def body(x_ref, wa_ref, wb_ref, b_ref, oa_ref, ob_ref):
        x_ = x_ref[...]
        oa_ref[...] = _dotT(x_, wa_ref[...])
        ob_ref[...] = _dotT(x_, wb_ref[...]) + b_ref[...]

    return pl.pallas_call(
        body,
        grid=(grid,),
        in_specs=[
            pl.BlockSpec((blk, D), lambda i: (i, 0)),
            pl.BlockSpec((D, D), lambda i: (0, 0)),
            pl.BlockSpec((D, D), lambda i: (0, 0)),
            pl.BlockSpec((1, D), lambda i: (0, 0)),
        ],
        out_specs=[
            pl.BlockSpec((blk, D), lambda i: (i, 0)),
            pl.BlockSpec((blk, D), lambda i: (i, 0)),
        ],
        out_shape=[
            jax.ShapeDtypeStruct((n, D), jnp.float32),
            jax.ShapeDtypeStruct((n, D), jnp.float32),
        ],
    )(x, wa, wb, bias_b)


def _tc_sig_project2(parts, xr, wa, wb, bias_b, blk, n):
    grid = n // blk

    def body(p_ref, xr_ref, wa_ref, wb_ref, b_ref, oa_ref, ob_ref):
        h = jax.nn.sigmoid(p_ref[0] + p_ref[1] + xr_ref[...])
        oa_ref[...] = _dotT(h, wa_ref[...])
        ob_ref[...] = _dotT(h, wb_ref[...]) + b_ref[...]

    return pl.pallas_call(
        body,
        grid=(grid,),
        in_specs=[
            pl.BlockSpec((NUM_CORES, blk, D), lambda i: (0, i, 0)),
            pl.BlockSpec((blk, D), lambda i: (i, 0)),
            pl.BlockSpec((D, D), lambda i: (0, 0)),
            pl.BlockSpec((D, D), lambda i: (0, 0)),
            pl.BlockSpec((1, D), lambda i: (0, 0)),
        ],
        out_specs=[
            pl.BlockSpec((blk, D), lambda i: (i, 0)),
            pl.BlockSpec((blk, D), lambda i: (i, 0)),
        ],
        out_shape=[
            jax.ShapeDtypeStruct((n, D), jnp.float32),
            jax.ShapeDtypeStruct((n, D), jnp.float32),
        ],
    )(parts, xr, wa, wb, bias_b)


def _tc_sig_sum(parts, hr, blk, n):
    grid = n // blk

    def body(p_ref, hr_ref, o_ref):
        o_ref[...] = jax.nn.sigmoid(p_ref[0] + p_ref[1] + hr_ref[...])

    return pl.pallas_call(
        body,
        grid=(grid,),
        in_specs=[
            pl.BlockSpec((NUM_CORES, blk, D), lambda i: (0, i, 0)),
            pl.BlockSpec((blk, D), lambda i: (i, 0)),
        ],
        out_specs=pl.BlockSpec((blk, D), lambda i: (i, 0)),
        out_shape=jax.ShapeDtypeStruct((n, D), jnp.float32),
    )(parts, hr)


def _sc_segment_sum(g, src_r, dst_r, w_r, n, ngrp):
    rows_per_sub = (n // NUM_SUBCORES) // 8 * 8
    rem_rows = n - NUM_SUBCORES * rows_per_sub
    mesh = plsc.VectorSubcoreMesh(core_axis_name="c", subcore_axis_name="s")

    @functools.partial(
        pl.kernel,
        mesh=mesh,
        out_type=jax.ShapeDtypeStruct((NUM_CORES, n, D), jnp.float32),
        scratch_types=[
            pltpu.VMEM((GRP, CHUNK), jnp.int32),
            pltpu.VMEM((GRP, CHUNK), jnp.int32),
            pltpu.VMEM((GRP * CHUNK,), jnp.float32),
            pltpu.VMEM((CHUNK, D), jnp.float32),
            pltpu.VMEM_SHARED((n, D), jnp.float32),
            pltpu.SemaphoreType.DMA,
        ],
    )
    def k(g_hbm, src_hbm, dst_hbm, w_hbm, out_hbm,
          src_v, dst_v, w_v, rows_v, agg_s, sem):
        cid = lax.axis_index("c")
        sid = lax.axis_index("s")
        wid = sid * NUM_CORES + cid

        def zbody(i, carry):
            for j in range(D // LANES):
                rows_v[i, pl.ds(j * LANES, LANES)] = jnp.zeros((LANES,), jnp.float32)
            return carry
        lax.fori_loop(0, CHUNK, zbody, 0)
        zbase = sid * rows_per_sub
        done = 0
        while done < rows_per_sub:
            step = min(CHUNK, rows_per_sub - done)
            pltpu.sync_copy(rows_v.at[pl.ds(0, step)],
                            agg_s.at[pl.ds(zbase + done, step)])
            done += step
        if rem_rows:
            @pl.when(sid == NUM_SUBCORES - 1)
            def _zero_tail():
                pltpu.sync_copy(
                    rows_v.at[pl.ds(0, rem_rows)],
                    agg_s.at[pl.ds(NUM_SUBCORES * rows_per_sub, rem_rows)])
        plsc.subcore_barrier()

        def grp_body(gg, carry):
            pltpu.sync_copy(src_hbm.at[wid, gg], src_v)
            pltpu.sync_copy(dst_hbm.at[wid, gg], dst_v)
            pltpu.sync_copy(w_hbm.at[wid, gg], w_v)

            def chunk_body(kk, c1):
                pltpu.async_copy(g_hbm.at[src_v.at[kk]], rows_v, sem).wait()

                def gbody(gi, c2):
                    w16 = w_v[pl.ds(kk * CHUNK + gi * LANES, LANES)]
                    for i in range(LANES):
                        w_splat = jnp.full((LANES,), w16[i], jnp.float32)
                        ei = gi * LANES + i
                        for j in range(D // LANES):
                            sl = pl.ds(j * LANES, LANES)
                            rows_v[ei, sl] = rows_v[ei, sl] * w_splat
                    return c2
                lax.fori_loop(0, CHUNK // LANES, gbody, 0)

                pltpu.sync_copy(rows_v, agg_s.at[dst_v.at[kk]], add=True)
                return c1
            lax.fori_loop(0, GRP, chunk_body, 0)
            return carry
        lax.fori_loop(0, ngrp, grp_body, 0)
        plsc.subcore_barrier()

        pltpu.sync_copy(agg_s.at[pl.ds(zbase, rows_per_sub)],
                        out_hbm.at[cid, pl.ds(zbase, rows_per_sub)])
        if rem_rows:
            @pl.when(sid == NUM_SUBCORES - 1)
            def _pub_tail():
                tb = NUM_SUBCORES * rows_per_sub
                pltpu.sync_copy(agg_s.at[pl.ds(tb, rem_rows)],
                                out_hbm.at[cid, pl.ds(tb, rem_rows)])

    return k(g, src_r, dst_r, w_r)


def kernel(X, edge_index, edge_weight,
           W_rel1, b_rel1, W_root1, b_root1,
           W_rel2, b_rel2, W_root2, b_root2):
    n = X.shape[0]
    e = edge_weight.shape[0]
    chunks_pt = -(-(-(-e // NW)) // CHUNK)
    chunks_pt = -(-chunks_pt // GRP) * GRP
    ngrp = chunks_pt // GRP
    e_pad = NW * chunks_pt * CHUNK
    pad = e_pad - e
    spread = (jnp.arange(pad, dtype=edge_index.dtype) * 7) % n
    src = jnp.concatenate([edge_index[0], spread])
    dst = jnp.concatenate([edge_index[1], spread])
    wf = jnp.concatenate([edge_weight, jnp.zeros((pad,), edge_weight.dtype)])

    blk = n // 10

    src_r = src.reshape(NW, ngrp, GRP, CHUNK)
    dst_r = dst.reshape(NW, ngrp, GRP, CHUNK)
    w_r = wf.reshape(NW, ngrp, GRP * CHUNK)
    b1 = (b_rel1 + b_root1).reshape(1, D)
    b2 = (b_rel2 + b_root2).reshape(1, D)

    xw1, xr1 = _tc_project2(X, W_rel1, W_root1, b1, blk, n)
    parts1 = _sc_segment_sum(xw1, src_r, dst_r, w_r, n, ngrp)
    hw2, hr2 = _tc_sig_project2(parts1, xr1, W_rel2, W_root2, b2, blk, n)
    parts2 = _sc_segment_sum(hw2, src_r, dst_r, w_r, n, ngrp)
    return _tc_sig_sum(parts2, hr2, blk, n)


# speedup vs baseline: 2.5989x; 1.2433x over previous
import functools

import jax
import jax.numpy as jnp
from jax import lax
from jax.experimental import pallas as pl
from jax.experimental.pallas import tpu as pltpu
from jax.experimental.pallas import tpu_sc as plsc

D = 128
LANES = 16
NUM_CORES = 2
NUM_SUBCORES = 16
NW = NUM_CORES * NUM_SUBCORES
CHUNK = 128
GRP = 16


def _dotT(x, w):
    return lax.dot_general(x, w, (((1,), (1,)), ((), ())),
                           preferred_element_type=jnp.float32)


def _tc_project2(x, wa, wb, bias_b, blk, n):
    grid = n // blk

    def body(x_ref, wa_ref, wb_ref, b_ref, oa_ref, ob_ref):
        x_ = x_ref[...]
        oa_ref[...] = _dotT(x_, wa_ref[...])
        ob_ref[...] = _dotT(x_, wb_ref[...]) + b_ref[...]

    return pl.pallas_call(
        body,
        grid=(grid,),
        in_specs=[
            pl.BlockSpec((blk, D), lambda i: (i, 0)),
            pl.BlockSpec((D, D), lambda i: (0, 0)),
            pl.BlockSpec((D, D), lambda i: (0, 0)),
            pl.BlockSpec((1, D), lambda i: (0, 0)),
        ],
        out_specs=[
            pl.BlockSpec((blk, D), lambda i: (i, 0)),
            pl.BlockSpec((blk, D), lambda i: (i, 0)),
        ],
        out_shape=[
            jax.ShapeDtypeStruct((n, D), jnp.float32),
            jax.ShapeDtypeStruct((n, D), jnp.float32),
        ],
    )(x, wa, wb, bias_b)


def _tc_sig_project2(parts, xr, wa, wb, bias_b, blk, n):
    grid = n // blk

    def body(p_ref, xr_ref, wa_ref, wb_ref, b_ref, oa_ref, ob_ref):
        h = jax.nn.sigmoid(p_ref[0] + p_ref[1] + xr_ref[...])
        oa_ref[...] = _dotT(h, wa_ref[...])
        ob_ref[...] = _dotT(h, wb_ref[...]) + b_ref[...]

    return pl.pallas_call(
        body,
        grid=(grid,),
        in_specs=[
            pl.BlockSpec((NUM_CORES, blk, D), lambda i: (0, i, 0)),
            pl.BlockSpec((blk, D), lambda i: (i, 0)),
            pl.BlockSpec((D, D), lambda i: (0, 0)),
            pl.BlockSpec((D, D), lambda i: (0, 0)),
            pl.BlockSpec((1, D), lambda i: (0, 0)),
        ],
        out_specs=[
            pl.BlockSpec((blk, D), lambda i: (i, 0)),
            pl.BlockSpec((blk, D), lambda i: (i, 0)),
        ],
        out_shape=[
            jax.ShapeDtypeStruct((n, D), jnp.float32),
            jax.ShapeDtypeStruct((n, D), jnp.float32),
        ],
    )(parts, xr, wa, wb, bias_b)


def _tc_sig_sum(parts, hr, blk, n):
    grid = n // blk

    def body(p_ref, hr_ref, o_ref):
        o_ref[...] = jax.nn.sigmoid(p_ref[0] + p_ref[1] + hr_ref[...])

    return pl.pallas_call(
        body,
        grid=(grid,),
        in_specs=[
            pl.BlockSpec((NUM_CORES, blk, D), lambda i: (0, i, 0)),
            pl.BlockSpec((blk, D), lambda i: (i, 0)),
        ],
        out_specs=pl.BlockSpec((blk, D), lambda i: (i, 0)),
        out_shape=jax.ShapeDtypeStruct((n, D), jnp.float32),
    )(parts, hr)


def _sc_segment_sum(g, src_r, dst_r, w_r, n, ngrp):
    rows_per_sub = (n // NUM_SUBCORES) // 8 * 8
    rem_rows = n - NUM_SUBCORES * rows_per_sub
    mesh = plsc.VectorSubcoreMesh(core_axis_name="c", subcore_axis_name="s")

    @functools.partial(
        pl.kernel,
        mesh=mesh,
        out_type=jax.ShapeDtypeStruct((NUM_CORES, n, D), jnp.float32),
        scratch_types=[
            pltpu.VMEM((GRP, CHUNK), jnp.int32),
            pltpu.VMEM((GRP, CHUNK), jnp.int32),
            pltpu.VMEM((GRP * CHUNK,), jnp.float32),
            pltpu.VMEM((CHUNK, D), jnp.float32),
            pltpu.VMEM_SHARED((n, D), jnp.float32),
            pltpu.SemaphoreType.DMA,
        ],
    )
    def k(g_hbm, src_hbm, dst_hbm, w_hbm, out_hbm,
          src_v, dst_v, w_v, rows_v, agg_s, sem):
        cid = lax.axis_index("c")
        sid = lax.axis_index("s")
        wid = sid * NUM_CORES + cid

        def zbody(i, carry):
            for j in range(D // LANES):
                rows_v[i, pl.ds(j * LANES, LANES)] = jnp.zeros((LANES,), jnp.float32)
            return carry
        lax.fori_loop(0, CHUNK, zbody, 0)
        zbase = sid * rows_per_sub
        done = 0
        while done < rows_per_sub:
            step = min(CHUNK, rows_per_sub - done)
            pltpu.sync_copy(rows_v.at[pl.ds(0, step)],
                            agg_s.at[pl.ds(zbase + done, step)])
            done += step
        if rem_rows:
            @pl.when(sid == NUM_SUBCORES - 1)
            def _zero_tail():
                pltpu.sync_copy(
                    rows_v.at[pl.ds(0, rem_rows)],
                    agg_s.at[pl.ds(NUM_SUBCORES * rows_per_sub, rem_rows)])
        plsc.subcore_barrier()

        def grp_body(gg, carry):
            pltpu.sync_copy(src_hbm.at[wid, gg], src_v)
            pltpu.sync_copy(dst_hbm.at[wid, gg], dst_v)
            pltpu.sync_copy(w_hbm.at[wid, gg], w_v)

            def chunk_body(kk, c1):
                pltpu.async_copy(g_hbm.at[src_v.at[kk]], rows_v, sem).wait()

                def gbody(gi, c2):
                    w16 = w_v[pl.ds(kk * CHUNK + gi * LANES, LANES)]
                    for i in range(LANES):
                        w_splat = jnp.full((LANES,), w16[i], jnp.float32)
                        ei = gi * LANES + i
                        for j in range(D // LANES):
                            sl = pl.ds(j * LANES, LANES)
                            rows_v[ei, sl] = rows_v[ei, sl] * w_splat
                    return c2
                lax.fori_loop(0, CHUNK // LANES, gbody, 0)

                pltpu.sync_copy(rows_v, agg_s.at[dst_v.at[kk]], add=True)
                return c1
            lax.fori_loop(0, GRP, chunk_body, 0)
            return carry
        lax.fori_loop(0, ngrp, grp_body, 0)
        plsc.subcore_barrier()

        pltpu.sync_copy(agg_s.at[pl.ds(zbase, rows_per_sub)],
                        out_hbm.at[cid, pl.ds(zbase, rows_per_sub)])
        if rem_rows:
            @pl.when(sid == NUM_SUBCORES - 1)
            def _pub_tail():
                tb = NUM_SUBCORES * rows_per_sub
                pltpu.sync_copy(agg_s.at[pl.ds(tb, rem_rows)],
                                out_hbm.at[cid, pl.ds(tb, rem_rows)])

    return k(g, src_r, dst_r, w_r)


def kernel(X, edge_index, edge_weight,
           W_rel1, b_rel1, W_root1, b_root1,
           W_rel2, b_rel2, W_root2, b_root2):
    n = X.shape[0]
    e = edge_weight.shape[0]
    chunks_pt = -(-(-(-e // NW)) // CHUNK)
    chunks_pt = -(-chunks_pt // GRP) * GRP
    ngrp = chunks_pt // GRP
    e_pad = NW * chunks_pt * CHUNK
    pad = e_pad - e
    spread = (jnp.arange(pad, dtype=edge_index.dtype) * 7) % n
    src = jnp.concatenate([edge_index[0], spread])
    dst = jnp.concatenate([edge_index[1], spread])
    wf = jnp.concatenate([edge_weight, jnp.zeros((pad,), edge_weight.dtype)])

    blk = n // 10

    src_r = src.reshape(NW, ngrp, GRP, CHUNK)
    dst_r = dst.reshape(NW, ngrp, GRP, CHUNK)
    w_r = wf.reshape(NW, ngrp, GRP * CHUNK)
    b1 = (b_rel1 + b_root1).reshape(1, D)
    b2 = (b_rel2 + b_root2).reshape(1, D)

    xw1, xr1 = _tc_project2(X, W_rel1, W_root1, b1, blk, n)
    parts1 = _sc_segment_sum(xw1, src_r, dst_r, w_r, n, ngrp)
    hw2, hr2 = _tc_sig_project2(parts1, xr1, W_rel2, W_root2, b2, blk, n)
    parts2 = _sc_segment_sum(hw2, src_r, dst_r, w_r, n, ngrp)
    return _tc_sig_sum(parts2, hr2, blk, n)


# speedup vs baseline: 2.9669x; 1.1416x over previous
import functools

import jax
import jax.numpy as jnp
from jax import lax
from jax.experimental import pallas as pl
from jax.experimental.pallas import tpu as pltpu
from jax.experimental.pallas import tpu_sc as plsc

D = 128
LANES = 16
NUM_CORES = 2
NUM_SUBCORES = 16
NW = NUM_CORES * NUM_SUBCORES
CHUNK = 128
GRP = 8


def _dotT(x, w):
    return lax.dot_general(x, w, (((1,), (1,)), ((), ())),
                           preferred_element_type=jnp.float32)


def _tc_project2(x, wa, wb, bias_b, blk, n):
    grid = n // blk

    def body(x_ref, wa_ref, wb_ref, b_ref, oa_ref, ob_ref):
        x_ = x_ref[...]
        oa_ref[...] = _dotT(x_, wa_ref[...])
        ob_ref[...] = _dotT(x_, wb_ref[...]) + b_ref[...]

    return pl.pallas_call(
        body,
        grid=(grid,),
        in_specs=[
            pl.BlockSpec((blk, D), lambda i: (i, 0)),
            pl.BlockSpec((D, D), lambda i: (0, 0)),
            pl.BlockSpec((D, D), lambda i: (0, 0)),
            pl.BlockSpec((1, D), lambda i: (0, 0)),
        ],
        out_specs=[
            pl.BlockSpec((blk, D), lambda i: (i, 0)),
            pl.BlockSpec((blk, D), lambda i: (i, 0)),
        ],
        out_shape=[
            jax.ShapeDtypeStruct((n, D), jnp.float32),
            jax.ShapeDtypeStruct((n, D), jnp.float32),
        ],
    )(x, wa, wb, bias_b)


def _tc_sig_project2(parts, xr, wa, wb, bias_b, blk, n):
    grid = n // blk

    def body(p_ref, xr_ref, wa_ref, wb_ref, b_ref, oa_ref, ob_ref):
        h = jax.nn.sigmoid(p_ref[0] + p_ref[1] + xr_ref[...])
        oa_ref[...] = _dotT(h, wa_ref[...])
        ob_ref[...] = _dotT(h, wb_ref[...]) + b_ref[...]

    return pl.pallas_call(
        body,
        grid=(grid,),
        in_specs=[
            pl.BlockSpec((NUM_CORES, blk, D), lambda i: (0, i, 0)),
            pl.BlockSpec((blk, D), lambda i: (i, 0)),
            pl.BlockSpec((D, D), lambda i: (0, 0)),
            pl.BlockSpec((D, D), lambda i: (0, 0)),
            pl.BlockSpec((1, D), lambda i: (0, 0)),
        ],
        out_specs=[
            pl.BlockSpec((blk, D), lambda i: (i, 0)),
            pl.BlockSpec((blk, D), lambda i: (i, 0)),
        ],
        out_shape=[
            jax.ShapeDtypeStruct((n, D), jnp.float32),
            jax.ShapeDtypeStruct((n, D), jnp.float32),
        ],
    )(parts, xr, wa, wb, bias_b)


def _tc_sig_sum(parts, hr, blk, n):
    grid = n // blk

    def body(p_ref, hr_ref, o_ref):
        o_ref[...] = jax.nn.sigmoid(p_ref[0] + p_ref[1] + hr_ref[...])

    return pl.pallas_call(
        body,
        grid=(grid,),
        in_specs=[
            pl.BlockSpec((NUM_CORES, blk, D), lambda i: (0, i, 0)),
            pl.BlockSpec((blk, D), lambda i: (i, 0)),
        ],
        out_specs=pl.BlockSpec((blk, D), lambda i: (i, 0)),
        out_shape=jax.ShapeDtypeStruct((n, D), jnp.float32),
    )(parts, hr)


def _sc_segment_sum(g, src_r, dst_r, w_r, n, ngrp):
    rows_per_sub = (n // NUM_SUBCORES) // 8 * 8
    rem_rows = n - NUM_SUBCORES * rows_per_sub
    mesh = plsc.VectorSubcoreMesh(core_axis_name="c", subcore_axis_name="s")

    @functools.partial(
        pl.kernel,
        mesh=mesh,
        out_type=jax.ShapeDtypeStruct((NUM_CORES, n, D), jnp.float32),
        scratch_types=[
            pltpu.VMEM((GRP, CHUNK), jnp.int32),
            pltpu.VMEM((GRP, CHUNK), jnp.int32),
            pltpu.VMEM((GRP * CHUNK,), jnp.float32),
            pltpu.VMEM((CHUNK, D), jnp.float32),
            pltpu.VMEM((CHUNK, D), jnp.float32),
            pltpu.VMEM_SHARED((n, D), jnp.float32),
            pltpu.SemaphoreType.DMA,
            pltpu.SemaphoreType.DMA,
        ],
    )
    def k(g_hbm, src_hbm, dst_hbm, w_hbm, out_hbm,
          src_v, dst_v, w_v, rows_v, buf_b, agg_s, sem, sem_b):
        cid = lax.axis_index("c")
        sid = lax.axis_index("s")
        wid = sid * NUM_CORES + cid

        def zbody(i, carry):
            for j in range(D // LANES):
                rows_v[i, pl.ds(j * LANES, LANES)] = jnp.zeros((LANES,), jnp.float32)
            return carry
        lax.fori_loop(0, CHUNK, zbody, 0)
        zbase = sid * rows_per_sub
        done = 0
        while done < rows_per_sub:
            step = min(CHUNK, rows_per_sub - done)
            pltpu.sync_copy(rows_v.at[pl.ds(0, step)],
                            agg_s.at[pl.ds(zbase + done, step)])
            done += step
        if rem_rows:
            @pl.when(sid == NUM_SUBCORES - 1)
            def _zero_tail():
                pltpu.sync_copy(
                    rows_v.at[pl.ds(0, rem_rows)],
                    agg_s.at[pl.ds(NUM_SUBCORES * rows_per_sub, rem_rows)])
        plsc.subcore_barrier()

        def _mul(rb, kk):
            def gbody(gi, c2):
                w16 = w_v[pl.ds(kk * CHUNK + gi * LANES, LANES)]
                for i in range(LANES):
                    w_splat = jnp.full((LANES,), w16[i], jnp.float32)
                    ei = gi * LANES + i
                    for j in range(D // LANES):
                        sl = pl.ds(j * LANES, LANES)
                        rb[ei, sl] = rb[ei, sl] * w_splat
                return c2
            lax.fori_loop(0, CHUNK // LANES, gbody, 0)

        def grp_body(gg, carry):
            pltpu.sync_copy(src_hbm.at[wid, gg], src_v)
            pltpu.sync_copy(dst_hbm.at[wid, gg], dst_v)
            pltpu.sync_copy(w_hbm.at[wid, gg], w_v)

            pltpu.async_copy(g_hbm.at[src_v.at[0]], rows_v, sem)

            def pair_body(t, c1):
                c0 = 2 * t
                pltpu.make_async_copy(g_hbm.at[src_v.at[c0]], rows_v,
                                      sem).wait()
                if True:
                    @pl.when(c0 > 0)
                    def _():
                        pltpu.sync_copy(buf_b, agg_s.at[dst_v.at[c0 - 1]],
                                        add=True)
                pltpu.async_copy(g_hbm.at[src_v.at[c0 + 1]], buf_b, sem_b)
                _mul(rows_v, c0)
                pltpu.make_async_copy(g_hbm.at[src_v.at[c0 + 1]], buf_b,
                                      sem_b).wait()
                pltpu.sync_copy(rows_v, agg_s.at[dst_v.at[c0]], add=True)

                @pl.when(c0 + 2 < GRP)
                def _():
                    pltpu.async_copy(g_hbm.at[src_v.at[c0 + 2]], rows_v, sem)
                _mul(buf_b, c0 + 1)
                return c1
            lax.fori_loop(0, GRP // 2, pair_body, 0)
            pltpu.sync_copy(buf_b, agg_s.at[dst_v.at[GRP - 1]], add=True)
            return carry
        lax.fori_loop(0, ngrp, grp_body, 0)
        plsc.subcore_barrier()

        pltpu.sync_copy(agg_s.at[pl.ds(zbase, rows_per_sub)],
                        out_hbm.at[cid, pl.ds(zbase, rows_per_sub)])
        if rem_rows:
            @pl.when(sid == NUM_SUBCORES - 1)
            def _pub_tail():
                tb = NUM_SUBCORES * rows_per_sub
                pltpu.sync_copy(agg_s.at[pl.ds(tb, rem_rows)],
                                out_hbm.at[cid, pl.ds(tb, rem_rows)])

    return k(g, src_r, dst_r, w_r)


def kernel(X, edge_index, edge_weight,
           W_rel1, b_rel1, W_root1, b_root1,
           W_rel2, b_rel2, W_root2, b_root2):
    n = X.shape[0]
    e = edge_weight.shape[0]
    chunks_pt = -(-(-(-e // NW)) // CHUNK)
    chunks_pt = -(-chunks_pt // GRP) * GRP
    ngrp = chunks_pt // GRP
    e_pad = NW * chunks_pt * CHUNK
    pad = e_pad - e
    spread = (jnp.arange(pad, dtype=edge_index.dtype) * 7) % n
    src = jnp.concatenate([edge_index[0], spread])
    dst = jnp.concatenate([edge_index[1], spread])
    wf = jnp.concatenate([edge_weight, jnp.zeros((pad,), edge_weight.dtype)])

    blk = n // 10

    src_r = src.reshape(NW, ngrp, GRP, CHUNK)
    dst_r = dst.reshape(NW, ngrp, GRP, CHUNK)
    w_r = wf.reshape(NW, ngrp, GRP * CHUNK)
    b1 = (b_rel1 + b_root1).reshape(1, D)
    b2 = (b_rel2 + b_root2).reshape(1, D)

    xw1, xr1 = _tc_project2(X, W_rel1, W_root1, b1, blk, n)
    parts1 = _sc_segment_sum(xw1, src_r, dst_r, w_r, n, ngrp)
    hw2, hr2 = _tc_sig_project2(parts1, xr1, W_rel2, W_root2, b2, blk, n)
    parts2 = _sc_segment_sum(hw2, src_r, dst_r, w_r, n, ngrp)
    return _tc_sig_sum(parts2, hr2, blk, n)


# speedup vs baseline: 3.0146x; 1.0161x over previous
import functools

import jax
import jax.numpy as jnp
from jax import lax
from jax.experimental import pallas as pl
from jax.experimental.pallas import tpu as pltpu
from jax.experimental.pallas import tpu_sc as plsc

D = 128
LANES = 16
NUM_CORES = 2
NUM_SUBCORES = 16
NW = NUM_CORES * NUM_SUBCORES
CHUNK = 128
GRP = 10


def _dotT(x, w):
    return lax.dot_general(x, w, (((1,), (1,)), ((), ())),
                           preferred_element_type=jnp.float32)


def _tc_project2(x, wa, wb, bias_b, blk, n):
    grid = n // blk

    def body(x_ref, wa_ref, wb_ref, b_ref, oa_ref, ob_ref):
        x_ = x_ref[...]
        oa_ref[...] = _dotT(x_, wa_ref[...])
        ob_ref[...] = _dotT(x_, wb_ref[...]) + b_ref[...]

    return pl.pallas_call(
        body,
        grid=(grid,),
        in_specs=[
            pl.BlockSpec((blk, D), lambda i: (i, 0)),
            pl.BlockSpec((D, D), lambda i: (0, 0)),
            pl.BlockSpec((D, D), lambda i: (0, 0)),
            pl.BlockSpec((1, D), lambda i: (0, 0)),
        ],
        out_specs=[
            pl.BlockSpec((blk, D), lambda i: (i, 0)),
            pl.BlockSpec((blk, D), lambda i: (i, 0)),
        ],
        out_shape=[
            jax.ShapeDtypeStruct((n, D), jnp.float32),
            jax.ShapeDtypeStruct((n, D), jnp.float32),
        ],
    )(x, wa, wb, bias_b)


def _tc_sig_project2(parts, xr, wa, wb, bias_b, blk, n):
    grid = n // blk

    def body(p_ref, xr_ref, wa_ref, wb_ref, b_ref, oa_ref, ob_ref):
        h = jax.nn.sigmoid(p_ref[0] + p_ref[1] + xr_ref[...])
        oa_ref[...] = _dotT(h, wa_ref[...])
        ob_ref[...] = _dotT(h, wb_ref[...]) + b_ref[...]

    return pl.pallas_call(
        body,
        grid=(grid,),
        in_specs=[
            pl.BlockSpec((NUM_CORES, blk, D), lambda i: (0, i, 0)),
            pl.BlockSpec((blk, D), lambda i: (i, 0)),
            pl.BlockSpec((D, D), lambda i: (0, 0)),
            pl.BlockSpec((D, D), lambda i: (0, 0)),
            pl.BlockSpec((1, D), lambda i: (0, 0)),
        ],
        out_specs=[
            pl.BlockSpec((blk, D), lambda i: (i, 0)),
            pl.BlockSpec((blk, D), lambda i: (i, 0)),
        ],
        out_shape=[
            jax.ShapeDtypeStruct((n, D), jnp.float32),
            jax.ShapeDtypeStruct((n, D), jnp.float32),
        ],
    )(parts, xr, wa, wb, bias_b)


def _tc_sig_sum(parts, hr, blk, n):
    grid = n // blk

    def body(p_ref, hr_ref, o_ref):
        o_ref[...] = jax.nn.sigmoid(p_ref[0] + p_ref[1] + hr_ref[...])

    return pl.pallas_call(
        body,
        grid=(grid,),
        in_specs=[
            pl.BlockSpec((NUM_CORES, blk, D), lambda i: (0, i, 0)),
            pl.BlockSpec((blk, D), lambda i: (i, 0)),
        ],
        out_specs=pl.BlockSpec((blk, D), lambda i: (i, 0)),
        out_shape=jax.ShapeDtypeStruct((n, D), jnp.float32),
    )(parts, hr)


def _sc_segment_sum(g, src_r, dst_r, w_r, n, ngrp):
    rows_per_sub = (n // NUM_SUBCORES) // 8 * 8
    rem_rows = n - NUM_SUBCORES * rows_per_sub
    mesh = plsc.VectorSubcoreMesh(core_axis_name="c", subcore_axis_name="s")

    @functools.partial(
        pl.kernel,
        mesh=mesh,
        out_type=jax.ShapeDtypeStruct((NUM_CORES, n, D), jnp.float32),
        scratch_types=[
            pltpu.VMEM((GRP, CHUNK), jnp.int32),
            pltpu.VMEM((GRP, CHUNK), jnp.int32),
            pltpu.VMEM((GRP * CHUNK,), jnp.float32),
            pltpu.VMEM((CHUNK, D), jnp.float32),
            pltpu.VMEM((CHUNK, D), jnp.float32),
            pltpu.VMEM_SHARED((n, D), jnp.float32),
            pltpu.SemaphoreType.DMA,
            pltpu.SemaphoreType.DMA,
        ],
    )
    def k(g_hbm, src_hbm, dst_hbm, w_hbm, out_hbm,
          src_v, dst_v, w_v, rows_v, buf_b, agg_s, sem, sem_b):
        cid = lax.axis_index("c")
        sid = lax.axis_index("s")
        wid = sid * NUM_CORES + cid

        def zbody(i, carry):
            for j in range(D // LANES):
                rows_v[i, pl.ds(j * LANES, LANES)] = jnp.zeros((LANES,), jnp.float32)
            return carry
        lax.fori_loop(0, CHUNK, zbody, 0)
        zbase = sid * rows_per_sub
        done = 0
        while done < rows_per_sub:
            step = min(CHUNK, rows_per_sub - done)
            pltpu.sync_copy(rows_v.at[pl.ds(0, step)],
                            agg_s.at[pl.ds(zbase + done, step)])
            done += step
        if rem_rows:
            @pl.when(sid == NUM_SUBCORES - 1)
            def _zero_tail():
                pltpu.sync_copy(
                    rows_v.at[pl.ds(0, rem_rows)],
                    agg_s.at[pl.ds(NUM_SUBCORES * rows_per_sub, rem_rows)])
        plsc.subcore_barrier()

        def _mul(rb, kk):
            def gbody(gi, c2):
                w16 = w_v[pl.ds(kk * CHUNK + gi * LANES, LANES)]
                for i in range(LANES):
                    w_splat = jnp.full((LANES,), w16[i], jnp.float32)
                    ei = gi * LANES + i
                    for j in range(D // LANES):
                        sl = pl.ds(j * LANES, LANES)
                        rb[ei, sl] = rb[ei, sl] * w_splat
                return c2
            lax.fori_loop(0, CHUNK // LANES, gbody, 0)

        def grp_body(gg, carry):
            pltpu.sync_copy(src_hbm.at[wid, gg], src_v)
            pltpu.sync_copy(dst_hbm.at[wid, gg], dst_v)
            pltpu.sync_copy(w_hbm.at[wid, gg], w_v)

            pltpu.async_copy(g_hbm.at[src_v.at[0]], rows_v, sem)

            def pair_body(t, c1):
                c0 = 2 * t
                pltpu.make_async_copy(g_hbm.at[src_v.at[c0]], rows_v,
                                      sem).wait()
                if True:
                    @pl.when(c0 > 0)
                    def _():
                        pltpu.sync_copy(buf_b, agg_s.at[dst_v.at[c0 - 1]],
                                        add=True)
                pltpu.async_copy(g_hbm.at[src_v.at[c0 + 1]], buf_b, sem_b)
                _mul(rows_v, c0)
                pltpu.make_async_copy(g_hbm.at[src_v.at[c0 + 1]], buf_b,
                                      sem_b).wait()
                pltpu.sync_copy(rows_v, agg_s.at[dst_v.at[c0]], add=True)

                @pl.when(c0 + 2 < GRP)
                def _():
                    pltpu.async_copy(g_hbm.at[src_v.at[c0 + 2]], rows_v, sem)
                _mul(buf_b, c0 + 1)
                return c1
            lax.fori_loop(0, GRP // 2, pair_body, 0)
            pltpu.sync_copy(buf_b, agg_s.at[dst_v.at[GRP - 1]], add=True)
            return carry
        lax.fori_loop(0, ngrp, grp_body, 0)
        plsc.subcore_barrier()

        pltpu.sync_copy(agg_s.at[pl.ds(zbase, rows_per_sub)],
                        out_hbm.at[cid, pl.ds(zbase, rows_per_sub)])
        if rem_rows:
            @pl.when(sid == NUM_SUBCORES - 1)
            def _pub_tail():
                tb = NUM_SUBCORES * rows_per_sub
                pltpu.sync_copy(agg_s.at[pl.ds(tb, rem_rows)],
                                out_hbm.at[cid, pl.ds(tb, rem_rows)])

    return k(g, src_r, dst_r, w_r)


def kernel(X, edge_index, edge_weight,
           W_rel1, b_rel1, W_root1, b_root1,
           W_rel2, b_rel2, W_root2, b_root2):
    n = X.shape[0]
    e = edge_weight.shape[0]
    chunks_pt = -(-(-(-e // NW)) // CHUNK)
    chunks_pt = -(-chunks_pt // GRP) * GRP
    ngrp = chunks_pt // GRP
    e_pad = NW * chunks_pt * CHUNK
    pad = e_pad - e
    spread = (jnp.arange(pad, dtype=edge_index.dtype) * 7) % n
    src = jnp.concatenate([edge_index[0], spread])
    dst = jnp.concatenate([edge_index[1], spread])
    wf = jnp.concatenate([edge_weight, jnp.zeros((pad,), edge_weight.dtype)])

    blk = n // 10

    src_r = src.reshape(NW, ngrp, GRP, CHUNK)
    dst_r = dst.reshape(NW, ngrp, GRP, CHUNK)
    w_r = wf.reshape(NW, ngrp, GRP * CHUNK)
    b1 = (b_rel1 + b_root1).reshape(1, D)
    b2 = (b_rel2 + b_root2).reshape(1, D)

    xw1, xr1 = _tc_project2(X, W_rel1, W_root1, b1, blk, n)
    parts1 = _sc_segment_sum(xw1, src_r, dst_r, w_r, n, ngrp)
    hw2, hr2 = _tc_sig_project2(parts1, xr1, W_rel2, W_root2, b2, blk, n)
    parts2 = _sc_segment_sum(hw2, src_r, dst_r, w_r, n, ngrp)
    return _tc_sig_sum(parts2, hr2, blk, n)
